# baseline (speedup 1.0000x reference)
import functools

import jax
import jax.numpy as jnp
from jax import lax
from jax.experimental import pallas as pl
from jax.experimental.pallas import tpu as pltpu
from jax.experimental.pallas import tpu_sc as plsc

N_EXPERTS = 8
N_GROUP = 4
GROUP_SZ = N_EXPERTS // N_GROUP
ROUTED_SCALE = 2.5
NEG_INF = float("-inf")
BIG = 1e9

TILE = 256
P_MAX = 4096 + N_EXPERTS * TILE
NT = P_MAX // TILE
NC = 2
NS = 16
NW = NC * NS


def _relu2(x):
    r = jnp.maximum(x, 0.0)
    return r * r


def _route_body(logits_t_ref, bias_ref, pos_lo_ref, pos_hi_ref,
                w_lo_ref, w_hi_ref, te_ref):
    lt = logits_t_ref[...]
    T = lt.shape[1]
    s = 1.0 / (1.0 + jnp.exp(-lt))
    sb = s + bias_ref[...]
    g = [sb[2 * i:2 * i + 1, :] + sb[2 * i + 1:2 * i + 2, :]
         for i in range(N_GROUP)]
    gsel = []
    for i in range(N_GROUP):
        rank = jnp.zeros_like(g[i], dtype=jnp.int32)
        for j in range(N_GROUP):
            if j == i:
                continue
            rank += ((g[j] > g[i]) | ((g[j] == g[i]) & (j < i))).astype(jnp.int32)
        gsel.append(rank < 2)
    ms = [jnp.where(gsel[e // GROUP_SZ], sb[e:e + 1, :], NEG_INF)
          for e in range(N_EXPERTS)]
    rows = []
    for i in range(N_EXPERTS):
        rank = jnp.zeros_like(ms[i], dtype=jnp.int32)
        for j in range(N_EXPERTS):
            if j == i:
                continue
            rank += ((ms[j] > ms[i]) | ((ms[j] == ms[i]) & (j < i))).astype(jnp.int32)
        rows.append((rank < 2).astype(jnp.float32) * s[i:i + 1, :])
    w = jnp.concatenate(rows, axis=0)
    denom = jnp.sum(w, axis=0, keepdims=True) + 1e-20
    comb = w * (ROUTED_SCALE / denom)

    sel = (comb > 0).astype(jnp.bfloat16)
    r_iota = lax.broadcasted_iota(jnp.int32, (T, T), 0)
    c_iota = lax.broadcasted_iota(jnp.int32, (T, T), 1)
    upper = (r_iota <= c_iota).astype(jnp.bfloat16)
    csum = jnp.dot(sel, upper, preferred_element_type=jnp.float32)
    cnt = csum[:, T - 1:T]
    cnt_i = cnt.astype(jnp.int32)
    pad_i = ((cnt_i + (TILE - 1)) >> 8) << 8
    pad_f = pad_i.astype(jnp.float32)
    e_r = lax.broadcasted_iota(jnp.int32, (N_EXPERTS, N_EXPERTS), 0)
    e_c = lax.broadcasted_iota(jnp.int32, (N_EXPERTS, N_EXPERTS), 1)
    strict_lower = (e_c < e_r).astype(jnp.float32)
    off = jnp.dot(strict_lower, pad_f,
                  preferred_element_type=jnp.float32)
    pos = off + csum - 1.0
    selb = comb > 0
    pos_lo = jnp.min(jnp.where(selb, pos, BIG), axis=0, keepdims=True)
    pos_hi = jnp.max(jnp.where(selb, pos, -1.0), axis=0, keepdims=True)
    pos_lo_ref[...] = pos_lo.astype(jnp.int32)
    pos_hi_ref[...] = pos_hi.astype(jnp.int32)
    w_lo_ref[...] = jnp.sum(jnp.where(selb & (pos == pos_lo), comb, 0.0),
                            axis=0, keepdims=True)
    w_hi_ref[...] = jnp.sum(jnp.where(selb & (pos == pos_hi), comb, 0.0),
                            axis=0, keepdims=True)
    bnd = off + pad_f
    t_start = (lax.broadcasted_iota(jnp.int32, (N_EXPERTS, NT), 1)
               * TILE).astype(jnp.float32)
    te = jnp.sum((t_start >= bnd).astype(jnp.int32), axis=0, keepdims=True)
    te_ref[...] = jnp.minimum(te, N_EXPERTS - 1)


def _routing(logits, gate_bias):
    T = logits.shape[0]
    outs = pl.pallas_call(
        _route_body,
        in_specs=[pl.BlockSpec((N_EXPERTS, T), lambda: (0, 0)),
                  pl.BlockSpec((N_EXPERTS, 1), lambda: (0, 0))],
        out_specs=[pl.BlockSpec((1, T), lambda: (0, 0)),
                   pl.BlockSpec((1, T), lambda: (0, 0)),
                   pl.BlockSpec((1, T), lambda: (0, 0)),
                   pl.BlockSpec((1, T), lambda: (0, 0)),
                   pl.BlockSpec((1, NT), lambda: (0, 0))],
        out_shape=[jax.ShapeDtypeStruct((1, T), jnp.int32),
                   jax.ShapeDtypeStruct((1, T), jnp.int32),
                   jax.ShapeDtypeStruct((1, T), jnp.float32),
                   jax.ShapeDtypeStruct((1, T), jnp.float32),
                   jax.ShapeDtypeStruct((1, NT), jnp.int32)],
    )(logits.T, gate_bias.reshape(N_EXPERTS, 1))
    pos_lo, pos_hi, w_lo, w_hi, te = outs
    return (pos_lo.reshape(T), pos_hi.reshape(T), w_lo.reshape(T),
            w_hi.reshape(T), te.reshape(NT))


def _scatter_build_body(pos_lo, pos_hi, w_lo, w_hi, gidx, gwt,
                        idx_ref, tok_ref, wv_ref, z_i_ref, z_f_ref, sem):
    cid = lax.axis_index("c")
    sid = lax.axis_index("s")

    @pl.when(cid == 0)
    def _():
        zchunk = P_MAX // NS
        for k in range(zchunk // 16):
            z_i_ref[pl.ds(16 * k, 16)] = jnp.zeros((16,), jnp.int32)
            z_f_ref[pl.ds(16 * k, 16)] = jnp.zeros((16,), jnp.float32)
        pltpu.sync_copy(z_i_ref, gidx.at[pl.ds(sid * zchunk, zchunk)])
        pltpu.sync_copy(z_f_ref, gwt.at[pl.ds(sid * zchunk, zchunk)])
        plsc.subcore_barrier()
        tb = sid * 128
        for k in range(8):
            tok_ref[pl.ds(16 * k, 16)] = tb + 16 * k + lax.iota(jnp.int32, 16)
        pltpu.sync_copy(pos_lo.at[pl.ds(tb, 128)], idx_ref.at[0])
        pltpu.sync_copy(pos_hi.at[pl.ds(tb, 128)], idx_ref.at[1])
        pltpu.sync_copy(w_lo.at[pl.ds(tb, 128)], wv_ref.at[0])
        pltpu.sync_copy(w_hi.at[pl.ds(tb, 128)], wv_ref.at[1])
        pltpu.async_copy(tok_ref, gidx.at[idx_ref.at[0]], sem).wait()
        pltpu.async_copy(tok_ref, gidx.at[idx_ref.at[1]], sem).wait()
        pltpu.async_copy(wv_ref.at[0], gwt.at[idx_ref.at[0]], sem).wait()
        pltpu.async_copy(wv_ref.at[1], gwt.at[idx_ref.at[1]], sem).wait()


def _scatter_build(pos_lo, pos_hi, w_lo, w_hi):
    mesh = plsc.VectorSubcoreMesh(core_axis_name="c", subcore_axis_name="s")
    f = pl.kernel(
        _scatter_build_body,
        out_type=[jax.ShapeDtypeStruct((P_MAX,), jnp.int32),
                  jax.ShapeDtypeStruct((P_MAX,), jnp.float32)],
        mesh=mesh,
        scratch_types=[pltpu.VMEM((2, 128), jnp.int32),
                       pltpu.VMEM((128,), jnp.int32),
                       pltpu.VMEM((2, 128), jnp.float32),
                       pltpu.VMEM((P_MAX // NS,), jnp.int32),
                       pltpu.VMEM((P_MAX // NS,), jnp.float32),
                       pltpu.SemaphoreType.DMA],
    )
    return f(pos_lo, pos_hi, w_lo, w_hi)


def _gather_body(hs, gidx, xg, idx_ref, rows_ref, sem):
    wid = lax.axis_index("s") * NC + lax.axis_index("c")
    per_w = P_MAX // NW
    for c in range(per_w // 64):
        base = wid * per_w + c * 64
        pltpu.sync_copy(gidx.at[pl.ds(base, 64)], idx_ref)
        pltpu.async_copy(hs.at[idx_ref], rows_ref, sem).wait()
        pltpu.sync_copy(rows_ref, xg.at[pl.ds(base, 64)])


def _gather(hs, gidx):
    T, D = hs.shape
    mesh = plsc.VectorSubcoreMesh(core_axis_name="c", subcore_axis_name="s")
    f = pl.kernel(
        _gather_body,
        out_type=jax.ShapeDtypeStruct((P_MAX, D), jnp.float32),
        mesh=mesh,
        scratch_types=[pltpu.VMEM((64,), jnp.int32),
                       pltpu.VMEM((64, D), jnp.float32),
                       pltpu.SemaphoreType.DMA],
    )
    return f(hs, gidx)


def _grouped_body(te_ref, xg_ref, gwt_ref, w1_ref, w2_ref, yg_ref):
    h = _relu2(jnp.dot(xg_ref[...], w1_ref[0],
                       preferred_element_type=jnp.float32))
    y = jnp.dot(h, w2_ref[0], preferred_element_type=jnp.float32)
    yg_ref[...] = y * gwt_ref[...]


def _grouped(te, xg, gwt, w1, w2):
    D = xg.shape[1]
    F = w1.shape[2]
    grid_spec = pltpu.PrefetchScalarGridSpec(
        num_scalar_prefetch=1,
        grid=(NT,),
        in_specs=[pl.BlockSpec((TILE, D), lambda i, te: (i, 0)),
                  pl.BlockSpec((TILE, 1), lambda i, te: (i, 0)),
                  pl.BlockSpec((1, D, F), lambda i, te: (te[i], 0, 0)),
                  pl.BlockSpec((1, F, D), lambda i, te: (te[i], 0, 0))],
        out_specs=pl.BlockSpec((TILE, D), lambda i, te: (i, 0)),
    )
    return pl.pallas_call(
        _grouped_body,
        grid_spec=grid_spec,
        out_shape=jax.ShapeDtypeStruct((P_MAX, D), jnp.float32),
    )(te, xg, gwt.reshape(P_MAX, 1), w1, w2)


def _shared_body(x_ref, ws1_ref, ws2_ref, out_ref):
    h = _relu2(jnp.dot(x_ref[...], ws1_ref[...],
                       preferred_element_type=jnp.float32))
    out_ref[...] = jnp.dot(h, ws2_ref[...], preferred_element_type=jnp.float32)


def _shared_mlp(hs, ws1, ws2, tm=1024):
    T, D = hs.shape
    SF = ws1.shape[1]
    return pl.pallas_call(
        _shared_body,
        grid=(T // tm,),
        in_specs=[pl.BlockSpec((tm, D), lambda m: (m, 0)),
                  pl.BlockSpec((D, SF), lambda m: (0, 0)),
                  pl.BlockSpec((SF, D), lambda m: (0, 0))],
        out_specs=pl.BlockSpec((tm, D), lambda m: (m, 0)),
        out_shape=jax.ShapeDtypeStruct((T, D), jnp.float32),
    )(hs, ws1, ws2)


def _combine_body(yg, shared, pos_lo, pos_hi, out,
                  il_ref, ih_ref, a_ref, b_ref, sem, sem2):
    wid = lax.axis_index("s") * NC + lax.axis_index("c")
    T = shared.shape[0]
    D = shared.shape[1]
    per_w = T // NW
    for c in range(per_w // 32):
        tb = wid * per_w + c * 32
        pltpu.sync_copy(pos_lo.at[pl.ds(tb, 32)], il_ref)
        pltpu.sync_copy(pos_hi.at[pl.ds(tb, 32)], ih_ref)
        cp_a = pltpu.async_copy(yg.at[il_ref], a_ref, sem)
        cp_b = pltpu.async_copy(yg.at[ih_ref], b_ref, sem2)
        cp_a.wait()
        cp_b.wait()

        def _add_rows(r, _):
            def _add16(q, _):
                a_ref[r, pl.ds(q * 16, 16)] = (a_ref[r, pl.ds(q * 16, 16)]
                                               + b_ref[r, pl.ds(q * 16, 16)])
                return ()
            lax.fori_loop(0, D // 16, _add16, (), unroll=8)
            return ()
        lax.fori_loop(0, 32, _add_rows, ())
        pltpu.sync_copy(shared.at[pl.ds(tb, 32)], b_ref)

        def _add_rows2(r, _):
            def _add16(q, _):
                a_ref[r, pl.ds(q * 16, 16)] = (a_ref[r, pl.ds(q * 16, 16)]
                                               + b_ref[r, pl.ds(q * 16, 16)])
                return ()
            lax.fori_loop(0, D // 16, _add16, (), unroll=8)
            return ()
        lax.fori_loop(0, 32, _add_rows2, ())
        pltpu.sync_copy(a_ref, out.at[pl.ds(tb, 32)])


def _combine(yg, shared, pos_lo, pos_hi):
    T, D = shared.shape
    mesh = plsc.VectorSubcoreMesh(core_axis_name="c", subcore_axis_name="s")
    f = pl.kernel(
        _combine_body,
        out_type=jax.ShapeDtypeStruct((T, D), jnp.float32),
        mesh=mesh,
        scratch_types=[pltpu.VMEM((32,), jnp.int32),
                       pltpu.VMEM((32,), jnp.int32),
                       pltpu.VMEM((32, D), jnp.float32),
                       pltpu.VMEM((32, D), jnp.float32),
                       pltpu.SemaphoreType.DMA,
                       pltpu.SemaphoreType.DMA],
    )
    return f(yg, shared, pos_lo, pos_hi)


def kernel(hidden_states, gate_w, gate_bias, w1, w2, ws1, ws2):
    logits = jnp.dot(hidden_states.astype(jnp.float32), gate_w.T)
    pos_lo, pos_hi, w_lo, w_hi, te = _routing(logits, gate_bias)
    gidx, gwt = _scatter_build(pos_lo, pos_hi, w_lo, w_hi)
    xg = _gather(hidden_states, gidx)
    yg = _grouped(te, xg, gwt, w1, w2)
    shared = _shared_mlp(hidden_states, ws1, ws2)
    return _combine(yg, shared, pos_lo, pos_hi)


# speedup vs baseline: 4.0784x; 4.0784x over previous
import functools

import jax
import jax.numpy as jnp
from jax.experimental import pallas as pl
from jax.experimental.pallas import tpu as pltpu

N_EXPERTS = 8
N_GROUP = 4
GROUP_SZ = N_EXPERTS // N_GROUP
ROUTED_SCALE = 2.5
NEG_INF = float("-inf")


def _relu2(x):
    r = jnp.maximum(x, 0.0)
    return r * r


def _route_body(logits_t_ref, bias_ref, comb_ref):
    lt = logits_t_ref[...]
    s = 1.0 / (1.0 + jnp.exp(-lt))
    sb = s + bias_ref[...]
    g = [sb[2 * i:2 * i + 1, :] + sb[2 * i + 1:2 * i + 2, :]
         for i in range(N_GROUP)]
    gsel = []
    for i in range(N_GROUP):
        rank = jnp.zeros_like(g[i], dtype=jnp.int32)
        for j in range(N_GROUP):
            if j == i:
                continue
            gt = g[j] > g[i]
            tie = (g[j] == g[i]) & (j < i)
            rank = rank + (gt | tie).astype(jnp.int32)
        gsel.append(rank < 2)
    ms = [jnp.where(gsel[e // GROUP_SZ], sb[e:e + 1, :], NEG_INF)
          for e in range(N_EXPERTS)]
    rows = []
    for i in range(N_EXPERTS):
        rank = jnp.zeros_like(ms[i], dtype=jnp.int32)
        for j in range(N_EXPERTS):
            if j == i:
                continue
            gt = ms[j] > ms[i]
            tie = (ms[j] == ms[i]) & (j < i)
            rank = rank + (gt | tie).astype(jnp.int32)
        rows.append((rank < 2).astype(jnp.float32) * s[i:i + 1, :])
    w = jnp.concatenate(rows, axis=0)
    denom = jnp.sum(w, axis=0, keepdims=True) + 1e-20
    wt = w * (ROUTED_SCALE / denom)
    comb_ref[...] = jax.lax.dot_general(
        wt, jnp.eye(N_EXPERTS, dtype=jnp.float32),
        (((0,), (0,)), ((), ())), preferred_element_type=jnp.float32)


def _moe_body(x_ref, comb_ref, w1_ref, w2_ref, ws1_ref, ws2_ref,
              out_ref):
    j = pl.program_id(1)

    @pl.when(j == 0)
    def _shared():
        x = x_ref[...]
        h = _relu2(jnp.dot(x, ws1_ref[...],
                           preferred_element_type=jnp.float32))
        out_ref[...] = jnp.dot(h, ws2_ref[...],
                               preferred_element_type=jnp.float32)

    @pl.when(j > 0)
    def _expert():
        lane = jax.lax.broadcasted_iota(jnp.int32, comb_ref.shape, 1)
        acc = None
        for k in range(2):
            e = 2 * (j - 1) + k
            h = _relu2(jnp.dot(x_ref[...], w1_ref[k],
                               preferred_element_type=jnp.float32))
            y = jnp.dot(h, w2_ref[k],
                        preferred_element_type=jnp.float32)
            ce = jnp.sum(jnp.where(lane == e, comb_ref[...], 0.0),
                         axis=1, keepdims=True)
            acc = ce * y if acc is None else acc + ce * y
        out_ref[...] += acc


@functools.partial(jax.jit, static_argnames=("tm",))
def _moe_fused(hidden_states, logits, gate_bias, w1, w2, ws1, ws2, tm=1024):
    T, D = hidden_states.shape
    E, _, F = w1.shape
    SF = ws1.shape[1]
    comb = pl.pallas_call(
        _route_body,
        in_specs=[pl.BlockSpec((N_EXPERTS, T), lambda: (0, 0)),
                  pl.BlockSpec((N_EXPERTS, 1), lambda: (0, 0))],
        out_specs=pl.BlockSpec((T, N_EXPERTS), lambda: (0, 0)),
        out_shape=jax.ShapeDtypeStruct((T, N_EXPERTS), jnp.float32),
    )(logits.T, gate_bias.reshape(N_EXPERTS, 1))
    grid = (T // tm, 1 + E // 2)
    return pl.pallas_call(
        _moe_body,
        grid=grid,
        in_specs=[
            pl.BlockSpec((tm, D), lambda m, j: (m, 0)),
            pl.BlockSpec((tm, N_EXPERTS), lambda m, j: (m, 0)),
            pl.BlockSpec((2, D, F), lambda m, j: (jnp.maximum(j, 1) - 1, 0, 0)),
            pl.BlockSpec((2, F, D), lambda m, j: (jnp.maximum(j, 1) - 1, 0, 0)),
            pl.BlockSpec((D, SF), lambda m, j: (0, 0)),
            pl.BlockSpec((SF, D), lambda m, j: (0, 0)),
        ],
        out_specs=pl.BlockSpec((tm, D), lambda m, j: (m, 0)),
        out_shape=jax.ShapeDtypeStruct((T, D), jnp.float32),
    )(hidden_states, comb, w1, w2, ws1, ws2)


def kernel(hidden_states, gate_w, gate_bias, w1, w2, ws1, ws2):
    logits = jnp.dot(hidden_states.astype(jnp.float32), gate_w.T)
    return _moe_fused(hidden_states, logits, gate_bias, w1, w2, ws1, ws2)


# speedup vs baseline: 4.0788x; 1.0001x over previous
import functools

import jax
import jax.numpy as jnp
from jax.experimental import pallas as pl
from jax.experimental.pallas import tpu as pltpu

N_EXPERTS = 8
N_GROUP = 4
GROUP_SZ = N_EXPERTS // N_GROUP
ROUTED_SCALE = 2.5
NEG_INF = float("-inf")


def _relu2(x):
    r = jnp.maximum(x, 0.0)
    return r * r


def _route_body(logits_t_ref, bias_ref, comb_ref):
    lt = logits_t_ref[...]
    s = 1.0 / (1.0 + jnp.exp(-lt))
    sb = s + bias_ref[...]
    g = [sb[2 * i:2 * i + 1, :] + sb[2 * i + 1:2 * i + 2, :]
         for i in range(N_GROUP)]
    gsel = []
    for i in range(N_GROUP):
        rank = jnp.zeros_like(g[i], dtype=jnp.int32)
        for j in range(N_GROUP):
            if j == i:
                continue
            gt = g[j] > g[i]
            tie = (g[j] == g[i]) & (j < i)
            rank = rank + (gt | tie).astype(jnp.int32)
        gsel.append(rank < 2)
    ms = [jnp.where(gsel[e // GROUP_SZ], sb[e:e + 1, :], NEG_INF)
          for e in range(N_EXPERTS)]
    rows = []
    for i in range(N_EXPERTS):
        rank = jnp.zeros_like(ms[i], dtype=jnp.int32)
        for j in range(N_EXPERTS):
            if j == i:
                continue
            gt = ms[j] > ms[i]
            tie = (ms[j] == ms[i]) & (j < i)
            rank = rank + (gt | tie).astype(jnp.int32)
        rows.append((rank < 2).astype(jnp.float32) * s[i:i + 1, :])
    w = jnp.concatenate(rows, axis=0)
    denom = jnp.sum(w, axis=0, keepdims=True) + 1e-20
    wt = w * (ROUTED_SCALE / denom)
    comb_ref[...] = jax.lax.dot_general(
        wt, jnp.eye(N_EXPERTS, dtype=jnp.float32),
        (((0,), (0,)), ((), ())), preferred_element_type=jnp.float32)


def _moe_body(x_ref, comb_ref, w1_ref, w2_ref, ws1_ref, ws2_ref,
              out_ref):
    j = pl.program_id(1)

    @pl.when(j == 0)
    def _shared():
        x = x_ref[...]
        h = _relu2(jnp.dot(x, ws1_ref[...],
                           preferred_element_type=jnp.float32))
        out_ref[...] = jnp.dot(h, ws2_ref[...],
                               preferred_element_type=jnp.float32)

    @pl.when(j > 0)
    def _expert():
        lane = jax.lax.broadcasted_iota(jnp.int32, comb_ref.shape, 1)
        acc = None
        for k in range(2):
            e = 2 * (j - 1) + k
            h = _relu2(jnp.dot(x_ref[...], w1_ref[k],
                               preferred_element_type=jnp.float32))
            y = jnp.dot(h, w2_ref[k],
                        preferred_element_type=jnp.float32)
            ce = jnp.sum(jnp.where(lane == e, comb_ref[...], 0.0),
                         axis=1, keepdims=True)
            acc = ce * y if acc is None else acc + ce * y
        out_ref[...] += acc


@functools.partial(jax.jit, static_argnames=("tm",))
def _moe_fused(hidden_states, logits, gate_bias, w1, w2, ws1, ws2, tm=1024):
    T, D = hidden_states.shape
    E, _, F = w1.shape
    SF = ws1.shape[1]
    comb = pl.pallas_call(
        _route_body,
        in_specs=[pl.BlockSpec((N_EXPERTS, T), lambda: (0, 0)),
                  pl.BlockSpec((N_EXPERTS, 1), lambda: (0, 0))],
        out_specs=pl.BlockSpec((T, N_EXPERTS), lambda: (0, 0)),
        out_shape=jax.ShapeDtypeStruct((T, N_EXPERTS), jnp.float32),
    )(logits.T, gate_bias.reshape(N_EXPERTS, 1))
    grid = (T // tm, 1 + E // 2)
    return pl.pallas_call(
        _moe_body,
        grid=grid,
        in_specs=[
            pl.BlockSpec((tm, D), lambda m, j: (m, 0)),
            pl.BlockSpec((tm, N_EXPERTS), lambda m, j: (m, 0)),
            pl.BlockSpec((2, D, F), lambda m, j: (jnp.maximum(j, 1) - 1, 0, 0)),
            pl.BlockSpec((2, F, D), lambda m, j: (jnp.maximum(j, 1) - 1, 0, 0)),
            pl.BlockSpec((D, SF), lambda m, j: (0, 0)),
            pl.BlockSpec((SF, D), lambda m, j: (0, 0)),
        ],
        out_specs=pl.BlockSpec((tm, D), lambda m, j: (m, 0)),
        out_shape=jax.ShapeDtypeStruct((T, D), jnp.float32),
        compiler_params=pltpu.CompilerParams(
            dimension_semantics=("parallel", "arbitrary")),
    )(hidden_states, comb, w1, w2, ws1, ws2)


def kernel(hidden_states, gate_w, gate_bias, w1, w2, ws1, ws2):
    logits = jnp.dot(hidden_states.astype(jnp.float32), gate_w.T)
    return _moe_fused(hidden_states, logits, gate_bias, w1, w2, ws1, ws2)
